# baseline (speedup 1.0000x reference)
import functools

import jax
import jax.numpy as jnp
from jax.experimental import pallas as pl
from jax.experimental.pallas import tpu as pltpu

TILE_M = 256
TILE_N = 8192


def _mm_kernel(x_ref, em_ref, o_ref):
    o_ref[...] = jax.lax.dot_general(
        x_ref[...], em_ref[...],
        dimension_numbers=(((1,), (1,)), ((), ())),
        preferred_element_type=jnp.float32,
    )


@functools.partial(jax.jit, static_argnames=())
def kernel(inputs, targets, em):
    del targets
    m, k = inputs.shape
    n = em.shape[0]
    grid = (m // TILE_M, pl.cdiv(n, TILE_N))
    out = pl.pallas_call(
        _mm_kernel,
        grid=grid,
        in_specs=[
            pl.BlockSpec((TILE_M, k), lambda i, j: (i, 0)),
            pl.BlockSpec((TILE_N, k), lambda i, j: (j, 0)),
        ],
        out_specs=pl.BlockSpec((TILE_M, TILE_N), lambda i, j: (i, j)),
        out_shape=jax.ShapeDtypeStruct((m, n), jnp.float32),
        compiler_params=pltpu.CompilerParams(
            dimension_semantics=("arbitrary", "arbitrary"),
        ),
    )(inputs, em)
    return out


# speedup vs baseline: 1.1947x; 1.1947x over previous
import functools

import jax
import jax.numpy as jnp
from jax.experimental import pallas as pl
from jax.experimental.pallas import tpu as pltpu

TILE_M = 32


def _mm_kernel(x_ref, emt_ref, o_ref):
    o_ref[...] = jax.lax.dot_general(
        x_ref[...], emt_ref[...],
        dimension_numbers=(((1,), (0,)), ((), ())),
        preferred_element_type=jnp.float32,
    )


@functools.partial(jax.jit, static_argnames=())
def kernel(inputs, targets, em):
    del targets
    m, k = inputs.shape
    n = em.shape[0]
    emt = em.T
    out = pl.pallas_call(
        _mm_kernel,
        grid=(m // TILE_M,),
        in_specs=[
            pl.BlockSpec((TILE_M, k), lambda i: (i, 0)),
            pl.BlockSpec((k, n), lambda i: (0, 0)),
        ],
        out_specs=pl.BlockSpec((TILE_M, n), lambda i: (i, 0)),
        out_shape=jax.ShapeDtypeStruct((m, n), jnp.float32),
        compiler_params=pltpu.CompilerParams(
            dimension_semantics=("arbitrary",),
        ),
    )(inputs, emt)
    return out
